# baseline (speedup 1.0000x reference)
import functools

import jax
import jax.numpy as jnp
import numpy as np
from jax import lax
from jax.experimental import pallas as pl
from jax.experimental.pallas import tpu as pltpu
from jax.experimental.pallas import tpu_sc as plsc

_SPECS = [(100000, i, 1) for i in range(20)] + [(100000, 20 + 10 * j, 10) for j in range(6)]
_NUM_TOKENS = 8
_CHUNK = 208
_NF = 80
_D = 64
_B = 4096
_NW = 32
_ROWS_PER_W = _B * _NF // _NW
_GCHUNK = 1024
_NCH = _ROWS_PER_W // _GCHUNK


def _sc_gather(idx_hbm, table_hbm, out_hbm, idx_v, gbuf, sem):
    wid = lax.axis_index("s") * 2 + lax.axis_index("c")
    for ch in range(_NCH):
        base = wid * (_NCH * 8) + ch * 8
        pltpu.sync_copy(idx_hbm.at[pl.ds(base, 8)], idx_v)
        cps = [
            pltpu.async_copy(table_hbm.at[idx_v.at[j]],
                             gbuf.at[pl.ds(j * 128, 128)], sem)
            for j in range(8)
        ]
        for cp in cps:
            cp.wait()
        pltpu.sync_copy(gbuf, out_hbm.at[pl.ds(base * 128, _GCHUNK)])


def _run_sc_gather(idx2d, table_flat):
    mesh = plsc.VectorSubcoreMesh(core_axis_name="c", subcore_axis_name="s")
    return pl.kernel(
        _sc_gather,
        mesh=mesh,
        compiler_params=pltpu.CompilerParams(use_tc_tiling_on_sc=False),
        out_type=jax.ShapeDtypeStruct((_B * _NF, _D), jnp.float32),
        scratch_types=[
            pltpu.VMEM((8, 128), jnp.int32),
            pltpu.VMEM((_GCHUNK, _D), jnp.float32),
            pltpu.SemaphoreType.DMA,
        ],
    )(idx2d, table_flat)


_BT = 256


def _tc_body(g_ref, intf_ref, w1_ref, b1_ref, w2_ref, b2_ref,
             pw_ref, pb_ref, lg_ref, lb_ref, out_ref):
    f32 = jnp.float32
    xi = intf_ref[...]
    nz = (xi != 0).astype(f32)
    c_iota = lax.broadcasted_iota(jnp.int32, (_NF, 6), 0)
    j_iota = lax.broadcasted_iota(jnp.int32, (_NF, 6), 1)
    G = ((c_iota >= 20 + 10 * j_iota) & (c_iota < 30 + 10 * j_iota)).astype(f32)
    counts = jnp.dot(nz, G, preferred_element_type=f32)
    recip = 1.0 / jnp.maximum(counts, 1.0)

    pieces = [g_ref[c] for c in range(20)]
    for j in range(6):
        s = g_ref[20 + 10 * j]
        for c in range(1, 10):
            s = s + g_ref[20 + 10 * j + c]
        pieces.append(s * recip[:, j:j + 1])
    cat = jnp.concatenate(pieces, axis=-1)

    h = jnp.dot(cat, w1_ref[...], preferred_element_type=f32) + b1_ref[...]
    h = h * jax.nn.sigmoid(h)
    gate = jax.nn.sigmoid(jnp.dot(h, w2_ref[...], preferred_element_type=f32) + b2_ref[...])
    cat = cat * gate * 2.0

    for t in range(_NUM_TOKENS):
        xt = cat[:, _CHUNK * t:_CHUNK * (t + 1)]
        y = jnp.dot(xt, pw_ref[t], preferred_element_type=f32) + pb_ref[t]
        mu = jnp.mean(y, axis=-1, keepdims=True)
        var = jnp.mean((y - mu) ** 2, axis=-1, keepdims=True)
        out_ref[:, t, :] = (y - mu) / jnp.sqrt(var + 1e-5) * lg_ref[t] + lb_ref[t]


def _run_tc(g3, int_feats, w1, b1, w2, b2, proj_w, proj_b, ln_g, ln_b):
    full = lambda shape: pl.BlockSpec(shape, lambda i: tuple(0 for _ in shape))
    grid = _B // _BT
    return pl.pallas_call(
        _tc_body,
        grid=(grid,),
        in_specs=[
            pl.BlockSpec((_NF, _BT, _D), lambda i: (0, i, 0)),
            pl.BlockSpec((_BT, _NF), lambda i: (i, 0)),
            full((1664, 416)),
            full((1, 416)),
            full((416, 1664)),
            full((1, 1664)),
            full((_NUM_TOKENS, _CHUNK, 256)),
            full((_NUM_TOKENS, 1, 256)),
            full((_NUM_TOKENS, 1, 256)),
            full((_NUM_TOKENS, 1, 256)),
        ],
        out_specs=pl.BlockSpec((_BT, _NUM_TOKENS, 256), lambda i: (i, 0, 0)),
        out_shape=jax.ShapeDtypeStruct((_B, _NUM_TOKENS, 256), jnp.float32),
        compiler_params=pltpu.CompilerParams(
            dimension_semantics=("arbitrary",),
        ),
    )(g3, int_feats, w1, b1, w2, b2, proj_w, proj_b, ln_g, ln_b)


def _flat_indices(int_feats):
    col_table = np.zeros((_NF,), dtype=np.int32)
    for i, (_, off, ln) in enumerate(_SPECS):
        col_table[off:off + ln] = i
    offsets = jnp.asarray(col_table * 100001, dtype=jnp.int32)
    flat = int_feats + offsets[None, :]
    return flat.T.reshape(_NF * _B // 128, 128)


def kernel(int_feats, tables, w1, b1, w2, b2, proj_w, proj_b, ln_g, ln_b):
    table_flat = tables.reshape(26 * 100001, _D)
    idx2d = _flat_indices(int_feats)
    rows = _run_sc_gather(idx2d, table_flat)
    g3 = rows.reshape(_NF, _B, _D)
    return _run_tc(
        g3, int_feats, w1,
        b1.reshape(1, 416), w2, b2.reshape(1, 1664),
        proj_w, proj_b.reshape(_NUM_TOKENS, 1, 256),
        ln_g.reshape(_NUM_TOKENS, 1, 256), ln_b.reshape(_NUM_TOKENS, 1, 256),
    )


# speedup vs baseline: 3.8608x; 3.8608x over previous
import functools

import jax
import jax.numpy as jnp
import numpy as np
from jax import lax
from jax.experimental import pallas as pl
from jax.experimental.pallas import tpu as pltpu
from jax.experimental.pallas import tpu_sc as plsc

_SPECS = [(100000, i, 1) for i in range(20)] + [(100000, 20 + 10 * j, 10) for j in range(6)]
_NUM_TOKENS = 8
_CHUNK = 208
_NF = 80
_NSEG = 26
_D = 64
_B = 4096
_NW = 32
_RPW = _B // _NW
_V = 100001
_PPS = 50176
_PCH = 512

_COL_TABLE = np.zeros((_NF,), dtype=np.int32)
for _i, (_, _off, _ln) in enumerate(_SPECS):
    _COL_TABLE[_off:_off + _ln] = _i


def _tr_body(a_ref, b_ref, out_ref):
    ya = a_ref[0].T
    yb = b_ref[0].T
    out_ref[0] = jnp.concatenate([ya, yb], axis=1)


def _run_transpose(t2):
    grid_p = _PPS // _PCH
    return pl.pallas_call(
        _tr_body,
        grid=(26, grid_p),
        in_specs=[
            pl.BlockSpec((1, _D, _PCH), lambda i, c: (i, 0, c)),
            pl.BlockSpec((1, _D, _PCH), lambda i, c: (i, 0, c + grid_p)),
        ],
        out_specs=pl.BlockSpec((1, _PCH, 128), lambda i, c: (i, c, 0)),
        out_shape=jax.ShapeDtypeStruct((26, _PPS, 128), jnp.float32),
        compiler_params=pltpu.CompilerParams(
            dimension_semantics=("arbitrary", "arbitrary"),
        ),
    )(t2, t2)


def _sc_body(idx_hbm, tab_hbm, out_hbm, idx_v, pidx_v, win_v, obuf_v, sem, semo):
    wid = lax.axis_index("s") * 2 + lax.axis_index("c")
    b0 = wid * _RPW

    def row_step(r, _):
        rr = r % 8
        pvecs = [idx_v[rr, pl.ds(16 * q, 16)] for q in range(5)]
        for q in range(5):
            pidx_v[pl.ds(16 * q, 16)] = pvecs[q] >> 1
        cp = pltpu.async_copy(tab_hbm.at[pidx_v], win_v, sem)
        cp.wait()
        hs = [pvecs[q] & 1 for q in range(5)]
        for c in range(20):
            h = hs[c // 16][c % 16]
            base = h * _D
            for m in range(4):
                obuf_v[(rr * 13) + c // 2, pl.ds((c % 2) * _D + 16 * m, 16)] = (
                    win_v[c, pl.ds(base + 16 * m, 16)])
        for j in range(6):
            accs = [None] * 4
            for t in range(10):
                c = 20 + 10 * j + t
                h = hs[c // 16][c % 16]
                base = h * _D
                for m in range(4):
                    piece = win_v[c, pl.ds(base + 16 * m, 16)]
                    accs[m] = piece if accs[m] is None else accs[m] + piece
            s = 20 + j
            for m in range(4):
                obuf_v[(rr * 13) + s // 2, pl.ds((s % 2) * _D + 16 * m, 16)] = accs[m]
        return ()

    def blk_step(blk, _):
        pltpu.sync_copy(idx_hbm.at[pl.ds(b0 + blk * 8, 8)], idx_v)
        lax.fori_loop(blk * 8, blk * 8 + 8, row_step, (), unroll=False)
        pltpu.sync_copy(obuf_v, out_hbm.at[pl.ds((b0 + blk * 8) * 13, 104)])
        return ()

    lax.fori_loop(0, _RPW // 8, blk_step, (), unroll=False)


def _run_sc(pk, tab):
    mesh = plsc.VectorSubcoreMesh(core_axis_name="c", subcore_axis_name="s")
    return pl.kernel(
        _sc_body,
        mesh=mesh,
        out_type=jax.ShapeDtypeStruct((_B * 13, 128), jnp.float32),
        scratch_types=[
            pltpu.VMEM((8, 128), jnp.int32),
            pltpu.VMEM((_NF,), jnp.int32),
            pltpu.VMEM((_NF, 128), jnp.float32),
            pltpu.VMEM((104, 128), jnp.float32),
            pltpu.SemaphoreType.DMA,
            pltpu.SemaphoreType.DMA,
        ],
    )(pk, tab)


_BT = 256


def _tc_body(cat_ref, intf_ref, w1_ref, b1_ref, w2_ref, b2_ref,
             pw_ref, pb_ref, lg_ref, lb_ref, out_ref):
    f32 = jnp.float32
    cat_sum = cat_ref[...]
    xi = intf_ref[...]
    nz = (xi != 0).astype(f32)
    c_iota = lax.broadcasted_iota(jnp.int32, (_NF, _NSEG), 0)
    s_iota = lax.broadcasted_iota(jnp.int32, (_NF, _NSEG), 1)
    H = ((s_iota >= 20) & (c_iota >= 10 * s_iota - 180)
         & (c_iota < 10 * s_iota - 170)).astype(f32)
    counts = jnp.dot(nz, H, preferred_element_type=f32)
    recip = 1.0 / jnp.maximum(counts, 1.0)
    seg_of = lax.broadcasted_iota(jnp.int32, (_NSEG, 1664), 1) // _D
    E = (seg_of == lax.broadcasted_iota(jnp.int32, (_NSEG, 1664), 0)).astype(f32)
    scale = jnp.dot(recip, E, preferred_element_type=f32)
    cat = cat_sum * scale

    h = jnp.dot(cat, w1_ref[...], preferred_element_type=f32) + b1_ref[...]
    h = h * jax.nn.sigmoid(h)
    gate = jax.nn.sigmoid(jnp.dot(h, w2_ref[...], preferred_element_type=f32) + b2_ref[...])
    cat = cat * gate * 2.0

    for t in range(_NUM_TOKENS):
        xt = cat[:, _CHUNK * t:_CHUNK * (t + 1)]
        y = jnp.dot(xt, pw_ref[t], preferred_element_type=f32) + pb_ref[t]
        mu = jnp.mean(y, axis=-1, keepdims=True)
        var = jnp.mean((y - mu) ** 2, axis=-1, keepdims=True)
        out_ref[:, t, :] = (y - mu) / jnp.sqrt(var + 1e-5) * lg_ref[t] + lb_ref[t]


def _run_tc(cat2d, int_feats, w1, b1, w2, b2, proj_w, proj_b, ln_g, ln_b):
    full = lambda shape: pl.BlockSpec(shape, lambda i: tuple(0 for _ in shape))
    return pl.pallas_call(
        _tc_body,
        grid=(_B // _BT,),
        in_specs=[
            pl.BlockSpec((_BT, _NSEG * _D), lambda i: (i, 0)),
            pl.BlockSpec((_BT, _NF), lambda i: (i, 0)),
            full((1664, 416)),
            full((1, 416)),
            full((416, 1664)),
            full((1, 1664)),
            full((_NUM_TOKENS, _CHUNK, 256)),
            full((_NUM_TOKENS, 1, 256)),
            full((_NUM_TOKENS, 1, 256)),
            full((_NUM_TOKENS, 1, 256)),
        ],
        out_specs=pl.BlockSpec((_BT, _NUM_TOKENS, 256), lambda i: (i, 0, 0)),
        out_shape=jax.ShapeDtypeStruct((_B, _NUM_TOKENS, 256), jnp.float32),
        compiler_params=pltpu.CompilerParams(
            dimension_semantics=("arbitrary",),
        ),
    )(cat2d, int_feats, w1, b1, w2, b2, proj_w, proj_b, ln_g, ln_b)


def _packed_indices(int_feats):
    offsets = jnp.asarray(_COL_TABLE.astype(np.int64) * (2 * _PPS), dtype=jnp.int32)
    h = (int_feats >= _PPS).astype(jnp.int32)
    p = int_feats - h * _PPS
    pk = offsets[None, :] + 2 * p + h
    return jnp.pad(pk, ((0, 0), (0, 128 - _NF)))


def kernel(int_feats, tables, w1, b1, w2, b2, proj_w, proj_b, ln_g, ln_b):
    t2 = tables.transpose(0, 2, 1)
    tab = _run_transpose(t2).reshape(26 * _PPS, 128)
    pk = _packed_indices(int_feats)
    cat_pk = _run_sc(pk, tab)
    return _run_tc(
        cat_pk.reshape(_B, _NSEG * _D), int_feats, w1,
        b1.reshape(1, 416), w2, b2.reshape(1, 1664),
        proj_w, proj_b.reshape(_NUM_TOKENS, 1, 256),
        ln_g.reshape(_NUM_TOKENS, 1, 256), ln_b.reshape(_NUM_TOKENS, 1, 256),
    )


# speedup vs baseline: 8.0605x; 2.0878x over previous
import functools

import jax
import jax.numpy as jnp
import numpy as np
from jax import lax
from jax.experimental import pallas as pl
from jax.experimental.pallas import tpu as pltpu
from jax.experimental.pallas import tpu_sc as plsc

_SPECS = [(100000, i, 1) for i in range(20)] + [(100000, 20 + 10 * j, 10) for j in range(6)]
_NUM_TOKENS = 8
_CHUNK = 208
_NF = 80
_NSEG = 26
_D = 64
_B = 4096
_NW = 32
_RPW = _B // _NW
_V = 100001
_PPS = 50176
_PCH = 3584

_COL_TABLE = np.zeros((_NF,), dtype=np.int32)
for _i, (_, _off, _ln) in enumerate(_SPECS):
    _COL_TABLE[_off:_off + _ln] = _i


def _tr_body(a_ref, b_ref, out_ref):
    f32 = jnp.float32
    eye = (lax.broadcasted_iota(jnp.int32, (_D, _D), 0)
           == lax.broadcasted_iota(jnp.int32, (_D, _D), 1)).astype(f32)
    dn = (((0,), (0,)), ((), ()))
    out_ref[0, :, 0:_D] = lax.dot_general(a_ref[0], eye, dn,
                                          preferred_element_type=f32)
    out_ref[0, :, _D:128] = lax.dot_general(b_ref[0], eye, dn,
                                            preferred_element_type=f32)


def _run_transpose(t2):
    grid_p = _PPS // _PCH
    return pl.pallas_call(
        _tr_body,
        grid=(26, grid_p),
        in_specs=[
            pl.BlockSpec((1, _D, _PCH), lambda i, c: (i, 0, c)),
            pl.BlockSpec((1, _D, _PCH), lambda i, c: (i, 0, c + grid_p)),
        ],
        out_specs=pl.BlockSpec((1, _PCH, 128), lambda i, c: (i, c, 0)),
        out_shape=jax.ShapeDtypeStruct((26, _PPS, 128), jnp.float32),
        compiler_params=pltpu.CompilerParams(
            dimension_semantics=("arbitrary", "arbitrary"),
        ),
    )(t2, t2)


def _sc_body(idx_hbm, tab_hbm, out_hbm, idx_v, pidx_v, win_v, obuf_v, sem, semo):
    wid = lax.axis_index("s") * 2 + lax.axis_index("c")
    b0 = wid * _RPW

    def row_step(r, _):
        rr = r % 8
        pvecs = [idx_v[rr, pl.ds(16 * q, 16)] for q in range(5)]
        for q in range(5):
            pidx_v[pl.ds(16 * q, 16)] = pvecs[q] >> 1
        cp = pltpu.async_copy(tab_hbm.at[pidx_v], win_v, sem)
        cp.wait()
        hs = [pvecs[q] & 1 for q in range(5)]
        for c in range(20):
            h = hs[c // 16][c % 16]
            base = h * _D
            for m in range(4):
                obuf_v[(rr * 13) + c // 2, pl.ds((c % 2) * _D + 16 * m, 16)] = (
                    win_v[c, pl.ds(base + 16 * m, 16)])
        for j in range(6):
            accs = [None] * 4
            for t in range(10):
                c = 20 + 10 * j + t
                h = hs[c // 16][c % 16]
                base = h * _D
                for m in range(4):
                    piece = win_v[c, pl.ds(base + 16 * m, 16)]
                    accs[m] = piece if accs[m] is None else accs[m] + piece
            s = 20 + j
            for m in range(4):
                obuf_v[(rr * 13) + s // 2, pl.ds((s % 2) * _D + 16 * m, 16)] = accs[m]
        return ()

    def blk_step(blk, _):
        pltpu.sync_copy(idx_hbm.at[pl.ds(b0 + blk * 8, 8)], idx_v)
        lax.fori_loop(blk * 8, blk * 8 + 8, row_step, (), unroll=False)
        pltpu.sync_copy(obuf_v, out_hbm.at[pl.ds((b0 + blk * 8) * 13, 104)])
        return ()

    lax.fori_loop(0, _RPW // 8, blk_step, (), unroll=False)


def _run_sc(pk, tab):
    mesh = plsc.VectorSubcoreMesh(core_axis_name="c", subcore_axis_name="s")
    return pl.kernel(
        _sc_body,
        mesh=mesh,
        out_type=jax.ShapeDtypeStruct((_B * 13, 128), jnp.float32),
        scratch_types=[
            pltpu.VMEM((8, 128), jnp.int32),
            pltpu.VMEM((_NF,), jnp.int32),
            pltpu.VMEM((_NF, 128), jnp.float32),
            pltpu.VMEM((104, 128), jnp.float32),
            pltpu.SemaphoreType.DMA,
            pltpu.SemaphoreType.DMA,
        ],
    )(pk, tab)


_BT = 256


def _tc_body(cat_ref, intf_ref, w1_ref, b1_ref, w2_ref, b2_ref,
             pw_ref, pb_ref, lg_ref, lb_ref, out_ref):
    f32 = jnp.float32
    cat_sum = cat_ref[...]
    xi = intf_ref[...]
    nz = (xi != 0).astype(f32)
    c_iota = lax.broadcasted_iota(jnp.int32, (_NF, _NSEG), 0)
    s_iota = lax.broadcasted_iota(jnp.int32, (_NF, _NSEG), 1)
    H = ((s_iota >= 20) & (c_iota >= 10 * s_iota - 180)
         & (c_iota < 10 * s_iota - 170)).astype(f32)
    counts = jnp.dot(nz, H, preferred_element_type=f32)
    recip = 1.0 / jnp.maximum(counts, 1.0)
    seg_of = lax.broadcasted_iota(jnp.int32, (_NSEG, 1664), 1) // _D
    E = (seg_of == lax.broadcasted_iota(jnp.int32, (_NSEG, 1664), 0)).astype(f32)
    scale = jnp.dot(recip, E, preferred_element_type=f32)
    cat = cat_sum * scale

    h = jnp.dot(cat, w1_ref[...], preferred_element_type=f32) + b1_ref[...]
    h = h * jax.nn.sigmoid(h)
    gate = jax.nn.sigmoid(jnp.dot(h, w2_ref[...], preferred_element_type=f32) + b2_ref[...])
    cat = cat * gate * 2.0

    for t in range(_NUM_TOKENS):
        xt = cat[:, _CHUNK * t:_CHUNK * (t + 1)]
        y = jnp.dot(xt, pw_ref[t], preferred_element_type=f32) + pb_ref[t]
        mu = jnp.mean(y, axis=-1, keepdims=True)
        var = jnp.mean((y - mu) ** 2, axis=-1, keepdims=True)
        out_ref[:, t, :] = (y - mu) / jnp.sqrt(var + 1e-5) * lg_ref[t] + lb_ref[t]


def _run_tc(cat2d, int_feats, w1, b1, w2, b2, proj_w, proj_b, ln_g, ln_b):
    full = lambda shape: pl.BlockSpec(shape, lambda i: tuple(0 for _ in shape))
    return pl.pallas_call(
        _tc_body,
        grid=(_B // _BT,),
        in_specs=[
            pl.BlockSpec((_BT, _NSEG * _D), lambda i: (i, 0)),
            pl.BlockSpec((_BT, _NF), lambda i: (i, 0)),
            full((1664, 416)),
            full((1, 416)),
            full((416, 1664)),
            full((1, 1664)),
            full((_NUM_TOKENS, _CHUNK, 256)),
            full((_NUM_TOKENS, 1, 256)),
            full((_NUM_TOKENS, 1, 256)),
            full((_NUM_TOKENS, 1, 256)),
        ],
        out_specs=pl.BlockSpec((_BT, _NUM_TOKENS, 256), lambda i: (i, 0, 0)),
        out_shape=jax.ShapeDtypeStruct((_B, _NUM_TOKENS, 256), jnp.float32),
        compiler_params=pltpu.CompilerParams(
            dimension_semantics=("arbitrary",),
        ),
    )(cat2d, int_feats, w1, b1, w2, b2, proj_w, proj_b, ln_g, ln_b)


def _packed_indices(int_feats):
    offsets = jnp.asarray(_COL_TABLE.astype(np.int64) * (2 * _PPS), dtype=jnp.int32)
    h = (int_feats >= _PPS).astype(jnp.int32)
    p = int_feats - h * _PPS
    pk = offsets[None, :] + 2 * p + h
    return jnp.pad(pk, ((0, 0), (0, 128 - _NF)))


def kernel(int_feats, tables, w1, b1, w2, b2, proj_w, proj_b, ln_g, ln_b):
    t2 = tables.transpose(0, 2, 1)
    tab = _run_transpose(t2).reshape(26 * _PPS, 128)
    pk = _packed_indices(int_feats)
    cat_pk = _run_sc(pk, tab)
    return _run_tc(
        cat_pk.reshape(_B, _NSEG * _D), int_feats, w1,
        b1.reshape(1, 416), w2, b2.reshape(1, 1664),
        proj_w, proj_b.reshape(_NUM_TOKENS, 1, 256),
        ln_g.reshape(_NUM_TOKENS, 1, 256), ln_b.reshape(_NUM_TOKENS, 1, 256),
    )


# speedup vs baseline: 11.6722x; 1.4481x over previous
import functools

import jax
import jax.numpy as jnp
import numpy as np
from jax import lax
from jax.experimental import pallas as pl
from jax.experimental.pallas import tpu as pltpu
from jax.experimental.pallas import tpu_sc as plsc

_SPECS = [(100000, i, 1) for i in range(20)] + [(100000, 20 + 10 * j, 10) for j in range(6)]
_NUM_TOKENS = 8
_CHUNK = 208
_NF = 80
_NSEG = 26
_D = 64
_B = 4096
_NW = 32
_RPW = _B // _NW
_V = 100001
_PPS = 50176
_PCH = 7168

_COL_TABLE = np.zeros((_NF,), dtype=np.int32)
for _i, (_, _off, _ln) in enumerate(_SPECS):
    _COL_TABLE[_off:_off + _ln] = _i


def _tr_body(a_ref, b_ref, out_ref):
    f32 = jnp.float32
    eye = (lax.broadcasted_iota(jnp.int32, (128, 128), 0)
           == lax.broadcasted_iota(jnp.int32, (128, 128), 1)).astype(f32)
    s = jnp.concatenate([a_ref[0], b_ref[0]], axis=0)
    out_ref[0] = lax.dot_general(s, eye, (((0,), (0,)), ((), ())),
                                 preferred_element_type=f32)


def _run_transpose(t2):
    grid_p = _PPS // _PCH
    return pl.pallas_call(
        _tr_body,
        grid=(26, grid_p),
        in_specs=[
            pl.BlockSpec((1, _D, _PCH), lambda i, c: (i, 0, c)),
            pl.BlockSpec((1, _D, _PCH), lambda i, c: (i, 0, c + grid_p)),
        ],
        out_specs=pl.BlockSpec((1, _PCH, 128), lambda i, c: (i, c, 0)),
        out_shape=jax.ShapeDtypeStruct((26, _PPS, 128), jnp.float32),
        compiler_params=pltpu.CompilerParams(
            dimension_semantics=("arbitrary", "arbitrary"),
        ),
    )(t2, t2)


def _sc_body(idx_hbm, tab_hbm, out_hbm, idx_v, pidx_v, win_v, obuf_v, sem, semo):
    wid = lax.axis_index("s") * 2 + lax.axis_index("c")
    b0 = wid * _RPW
    pltpu.sync_copy(idx_hbm.at[pl.ds(b0, _RPW)], idx_v)

    def issue(r, buf):
        for q in range(5):
            pidx_v[buf, pl.ds(16 * q, 16)] = idx_v[r, pl.ds(16 * q, 16)] >> 1
        pltpu.async_copy(tab_hbm.at[pidx_v.at[buf]], win_v.at[buf], sem)

    def wait_buf(buf):
        pltpu.make_async_copy(tab_hbm.at[pidx_v.at[buf]], win_v.at[buf], sem).wait()

    def process(r, rr, buf):
        hs = [idx_v[r, pl.ds(16 * q, 16)] & 1 for q in range(5)]
        for c in range(20):
            h = hs[c // 16][c % 16]
            base = h * _D
            for m in range(4):
                obuf_v[(rr * 13) + c // 2, pl.ds((c % 2) * _D + 16 * m, 16)] = (
                    win_v[buf, c, pl.ds(base + 16 * m, 16)])
        for j in range(6):
            accs = [None] * 4
            for t in range(10):
                c = 20 + 10 * j + t
                h = hs[c // 16][c % 16]
                base = h * _D
                for m in range(4):
                    piece = win_v[buf, c, pl.ds(base + 16 * m, 16)]
                    accs[m] = piece if accs[m] is None else accs[m] + piece
            s = 20 + j
            for m in range(4):
                obuf_v[(rr * 13) + s // 2, pl.ds((s % 2) * _D + 16 * m, 16)] = accs[m]

    issue(0, 0)

    def blk_step(blk, _):
        r0 = blk * 8
        for gg in range(4):
            ra = r0 + 2 * gg
            rb = ra + 1
            issue(rb, 1)
            wait_buf(0)
            process(ra, 2 * gg, 0)
            nxt = jnp.minimum(ra + 2, _RPW - 1)
            issue(nxt, 0)
            wait_buf(1)
            process(rb, 2 * gg + 1, 1)
        pltpu.sync_copy(obuf_v, out_hbm.at[pl.ds((b0 + r0) * 13, 104)])
        return ()

    lax.fori_loop(0, _RPW // 8, blk_step, (), unroll=False)
    wait_buf(0)


def _run_sc(pk, tab):
    mesh = plsc.VectorSubcoreMesh(core_axis_name="c", subcore_axis_name="s")
    return pl.kernel(
        _sc_body,
        mesh=mesh,
        out_type=jax.ShapeDtypeStruct((_B * 13, 128), jnp.float32),
        scratch_types=[
            pltpu.VMEM((_RPW, 128), jnp.int32),
            pltpu.VMEM((2, _NF), jnp.int32),
            pltpu.VMEM((2, _NF, 128), jnp.float32),
            pltpu.VMEM((104, 128), jnp.float32),
            pltpu.SemaphoreType.DMA,
            pltpu.SemaphoreType.DMA,
        ],
    )(pk, tab)


_BT = 256


def _tc_body(cat_ref, intf_ref, w1_ref, b1_ref, w2_ref, b2_ref,
             pw_ref, pb_ref, lg_ref, lb_ref, out_ref):
    f32 = jnp.float32
    cat_sum = cat_ref[...]
    xi = intf_ref[...]
    nz = (xi != 0).astype(f32)
    c_iota = lax.broadcasted_iota(jnp.int32, (_NF, _NSEG), 0)
    s_iota = lax.broadcasted_iota(jnp.int32, (_NF, _NSEG), 1)
    H = ((s_iota >= 20) & (c_iota >= 10 * s_iota - 180)
         & (c_iota < 10 * s_iota - 170)).astype(f32)
    counts = jnp.dot(nz, H, preferred_element_type=f32)
    recip = 1.0 / jnp.maximum(counts, 1.0)
    seg_of = lax.broadcasted_iota(jnp.int32, (_NSEG, 1664), 1) // _D
    E = (seg_of == lax.broadcasted_iota(jnp.int32, (_NSEG, 1664), 0)).astype(f32)
    scale = jnp.dot(recip, E, preferred_element_type=f32)
    cat = cat_sum * scale

    h = jnp.dot(cat, w1_ref[...], preferred_element_type=f32) + b1_ref[...]
    h = h * jax.nn.sigmoid(h)
    gate = jax.nn.sigmoid(jnp.dot(h, w2_ref[...], preferred_element_type=f32) + b2_ref[...])
    cat = cat * gate * 2.0

    for t in range(_NUM_TOKENS):
        xt = cat[:, _CHUNK * t:_CHUNK * (t + 1)]
        y = jnp.dot(xt, pw_ref[t], preferred_element_type=f32) + pb_ref[t]
        mu = jnp.mean(y, axis=-1, keepdims=True)
        var = jnp.mean((y - mu) ** 2, axis=-1, keepdims=True)
        out_ref[:, t, :] = (y - mu) / jnp.sqrt(var + 1e-5) * lg_ref[t] + lb_ref[t]


def _run_tc(cat2d, int_feats, w1, b1, w2, b2, proj_w, proj_b, ln_g, ln_b):
    full = lambda shape: pl.BlockSpec(shape, lambda i: tuple(0 for _ in shape))
    return pl.pallas_call(
        _tc_body,
        grid=(_B // _BT,),
        in_specs=[
            pl.BlockSpec((_BT, _NSEG * _D), lambda i: (i, 0)),
            pl.BlockSpec((_BT, _NF), lambda i: (i, 0)),
            full((1664, 416)),
            full((1, 416)),
            full((416, 1664)),
            full((1, 1664)),
            full((_NUM_TOKENS, _CHUNK, 256)),
            full((_NUM_TOKENS, 1, 256)),
            full((_NUM_TOKENS, 1, 256)),
            full((_NUM_TOKENS, 1, 256)),
        ],
        out_specs=pl.BlockSpec((_BT, _NUM_TOKENS, 256), lambda i: (i, 0, 0)),
        out_shape=jax.ShapeDtypeStruct((_B, _NUM_TOKENS, 256), jnp.float32),
        compiler_params=pltpu.CompilerParams(
            dimension_semantics=("arbitrary",),
        ),
    )(cat2d, int_feats, w1, b1, w2, b2, proj_w, proj_b, ln_g, ln_b)


def _packed_indices(int_feats):
    offsets = jnp.asarray(_COL_TABLE.astype(np.int64) * (2 * _PPS), dtype=jnp.int32)
    h = (int_feats >= _PPS).astype(jnp.int32)
    p = int_feats - h * _PPS
    pk = offsets[None, :] + 2 * p + h
    return jnp.pad(pk, ((0, 0), (0, 128 - _NF)))


def kernel(int_feats, tables, w1, b1, w2, b2, proj_w, proj_b, ln_g, ln_b):
    t2 = tables.transpose(0, 2, 1)
    tab = _run_transpose(t2).reshape(26 * _PPS, 128)
    pk = _packed_indices(int_feats)
    cat_pk = _run_sc(pk, tab)
    return _run_tc(
        cat_pk.reshape(_B, _NSEG * _D), int_feats, w1,
        b1.reshape(1, 416), w2, b2.reshape(1, 1664),
        proj_w, proj_b.reshape(_NUM_TOKENS, 1, 256),
        ln_g.reshape(_NUM_TOKENS, 1, 256), ln_b.reshape(_NUM_TOKENS, 1, 256),
    )


# speedup vs baseline: 13.0984x; 1.1222x over previous
import functools

import jax
import jax.numpy as jnp
import numpy as np
from jax import lax
from jax.experimental import pallas as pl
from jax.experimental.pallas import tpu as pltpu
from jax.experimental.pallas import tpu_sc as plsc

_SPECS = [(100000, i, 1) for i in range(20)] + [(100000, 20 + 10 * j, 10) for j in range(6)]
_NUM_TOKENS = 8
_CHUNK = 208
_NF = 80
_NSEG = 26
_D = 64
_B = 4096
_NW = 32
_RPW = _B // _NW
_V = 100001
_QPS = 25088
_PCH = 3584

_COL_TABLE = np.zeros((_NF,), dtype=np.int32)
for _i, (_, _off, _ln) in enumerate(_SPECS):
    _COL_TABLE[_off:_off + _ln] = _i


def _tr_body(a_ref, b_ref, c_ref, d_ref, out_ref):
    f32 = jnp.float32
    r = lax.broadcasted_iota(jnp.int32, (256, 256), 0)
    c = lax.broadcasted_iota(jnp.int32, (256, 256), 1)
    cm = c % 128
    perm = ((r // _D == cm // 32)
            & (r % _D == cm % 32 + 32 * (c // 128))).astype(f32)
    s = jnp.concatenate([a_ref[0], b_ref[0], c_ref[0], d_ref[0]], axis=0)
    s = jnp.where(jnp.isfinite(s), s, 0.0)
    y = lax.dot_general(s, perm, (((0,), (0,)), ((), ())),
                        preferred_element_type=f32)
    blo = lax.bitcast_convert_type(y[:, :128], jnp.int32)
    bhi = lax.bitcast_convert_type(y[:, 128:], jnp.int32)
    rlo = blo + 0x7FFF + ((blo >> 16) & 1)
    rhi = bhi + 0x7FFF + ((bhi >> 16) & 1)
    out_ref[0] = ((rlo >> 16) & 0xFFFF) | (rhi & jnp.int32(-65536))


def _run_transpose(t2):
    grid_p = _QPS // _PCH
    return pl.pallas_call(
        _tr_body,
        grid=(26, grid_p),
        in_specs=[
            pl.BlockSpec((1, _D, _PCH), lambda i, c: (i, 0, c)),
            pl.BlockSpec((1, _D, _PCH), lambda i, c: (i, 0, c + grid_p)),
            pl.BlockSpec((1, _D, _PCH), lambda i, c: (i, 0, c + 2 * grid_p)),
            pl.BlockSpec((1, _D, _PCH), lambda i, c: (i, 0, c + 3 * grid_p)),
        ],
        out_specs=pl.BlockSpec((1, _PCH, 128), lambda i, c: (i, c, 0)),
        out_shape=jax.ShapeDtypeStruct((26, _QPS, 128), jnp.int32),
        compiler_params=pltpu.CompilerParams(
            dimension_semantics=("arbitrary", "arbitrary"),
        ),
    )(t2, t2, t2, t2)


def _sc_body(idx_hbm, tab_hbm, out_hbm, idx_v, pidx_v, win_v, obuf_v, sem, semo):
    wid = lax.axis_index("s") * 2 + lax.axis_index("c")
    b0 = wid * _RPW
    pltpu.sync_copy(idx_hbm.at[pl.ds(b0, _RPW)], idx_v)

    def issue(r, buf):
        for q in range(5):
            pidx_v[buf, pl.ds(16 * q, 16)] = idx_v[r, pl.ds(16 * q, 16)] >> 2
        pltpu.async_copy(tab_hbm.at[pidx_v.at[buf]], win_v.at[buf], sem)

    def wait_buf(buf):
        pltpu.make_async_copy(tab_hbm.at[pidx_v.at[buf]], win_v.at[buf], sem).wait()

    def halves(buf, c, h):
        base = h * 32
        lo0 = win_v[buf, c, pl.ds(base, 16)]
        lo1 = win_v[buf, c, pl.ds(base + 16, 16)]
        return [
            lax.bitcast_convert_type(lo0 << 16, jnp.float32),
            lax.bitcast_convert_type(lo1 << 16, jnp.float32),
            lax.bitcast_convert_type(lo0 & -65536, jnp.float32),
            lax.bitcast_convert_type(lo1 & -65536, jnp.float32),
        ]

    def process(r, rr, buf):
        hs = [idx_v[r, pl.ds(16 * q, 16)] & 3 for q in range(5)]
        for c in range(20):
            h = hs[c // 16][c % 16]
            vals = halves(buf, c, h)
            for m in range(4):
                obuf_v[(rr * 13) + c // 2, pl.ds((c % 2) * _D + 16 * m, 16)] = vals[m]
        for j in range(6):
            accs = [None] * 4
            for t in range(10):
                c = 20 + 10 * j + t
                h = hs[c // 16][c % 16]
                vals = halves(buf, c, h)
                for m in range(4):
                    accs[m] = vals[m] if accs[m] is None else accs[m] + vals[m]
            s = 20 + j
            for m in range(4):
                obuf_v[(rr * 13) + s // 2, pl.ds((s % 2) * _D + 16 * m, 16)] = accs[m]

    issue(0, 0)

    def blk_step(blk, _):
        r0 = blk * 8
        for gg in range(4):
            ra = r0 + 2 * gg
            rb = ra + 1
            issue(rb, 1)
            wait_buf(0)
            process(ra, 2 * gg, 0)
            nxt = jnp.minimum(ra + 2, _RPW - 1)
            issue(nxt, 0)
            wait_buf(1)
            process(rb, 2 * gg + 1, 1)
        pltpu.sync_copy(obuf_v, out_hbm.at[pl.ds((b0 + r0) * 13, 104)])
        return ()

    lax.fori_loop(0, _RPW // 8, blk_step, (), unroll=False)
    wait_buf(0)


def _run_sc(pk, tab):
    mesh = plsc.VectorSubcoreMesh(core_axis_name="c", subcore_axis_name="s")
    return pl.kernel(
        _sc_body,
        mesh=mesh,
        out_type=jax.ShapeDtypeStruct((_B * 13, 128), jnp.float32),
        scratch_types=[
            pltpu.VMEM((_RPW, 128), jnp.int32),
            pltpu.VMEM((2, _NF), jnp.int32),
            pltpu.VMEM((2, _NF, 128), jnp.int32),
            pltpu.VMEM((104, 128), jnp.float32),
            pltpu.SemaphoreType.DMA,
            pltpu.SemaphoreType.DMA,
        ],
    )(pk, tab)


_BT = 256


def _tc_body(cat_ref, intf_ref, w1_ref, b1_ref, w2_ref, b2_ref,
             pw_ref, pb_ref, lg_ref, lb_ref, out_ref):
    f32 = jnp.float32
    cat_sum = cat_ref[...]
    xi = intf_ref[...]
    nz = (xi != 0).astype(f32)
    c_iota = lax.broadcasted_iota(jnp.int32, (_NF, _NSEG), 0)
    s_iota = lax.broadcasted_iota(jnp.int32, (_NF, _NSEG), 1)
    H = ((s_iota >= 20) & (c_iota >= 10 * s_iota - 180)
         & (c_iota < 10 * s_iota - 170)).astype(f32)
    counts = jnp.dot(nz, H, preferred_element_type=f32)
    recip = 1.0 / jnp.maximum(counts, 1.0)
    seg_of = lax.broadcasted_iota(jnp.int32, (_NSEG, 1664), 1) // _D
    E = (seg_of == lax.broadcasted_iota(jnp.int32, (_NSEG, 1664), 0)).astype(f32)
    scale = jnp.dot(recip, E, preferred_element_type=f32)
    cat = cat_sum * scale

    h = jnp.dot(cat, w1_ref[...], preferred_element_type=f32) + b1_ref[...]
    h = h * jax.nn.sigmoid(h)
    gate = jax.nn.sigmoid(jnp.dot(h, w2_ref[...], preferred_element_type=f32) + b2_ref[...])
    cat = cat * gate * 2.0

    for t in range(_NUM_TOKENS):
        xt = cat[:, _CHUNK * t:_CHUNK * (t + 1)]
        y = jnp.dot(xt, pw_ref[t], preferred_element_type=f32) + pb_ref[t]
        mu = jnp.mean(y, axis=-1, keepdims=True)
        var = jnp.mean((y - mu) ** 2, axis=-1, keepdims=True)
        out_ref[:, t, :] = (y - mu) / jnp.sqrt(var + 1e-5) * lg_ref[t] + lb_ref[t]


def _run_tc(cat2d, int_feats, w1, b1, w2, b2, proj_w, proj_b, ln_g, ln_b):
    full = lambda shape: pl.BlockSpec(shape, lambda i: tuple(0 for _ in shape))
    return pl.pallas_call(
        _tc_body,
        grid=(_B // _BT,),
        in_specs=[
            pl.BlockSpec((_BT, _NSEG * _D), lambda i: (i, 0)),
            pl.BlockSpec((_BT, _NF), lambda i: (i, 0)),
            full((1664, 416)),
            full((1, 416)),
            full((416, 1664)),
            full((1, 1664)),
            full((_NUM_TOKENS, _CHUNK, 256)),
            full((_NUM_TOKENS, 1, 256)),
            full((_NUM_TOKENS, 1, 256)),
            full((_NUM_TOKENS, 1, 256)),
        ],
        out_specs=pl.BlockSpec((_BT, _NUM_TOKENS, 256), lambda i: (i, 0, 0)),
        out_shape=jax.ShapeDtypeStruct((_B, _NUM_TOKENS, 256), jnp.float32),
        compiler_params=pltpu.CompilerParams(
            dimension_semantics=("arbitrary",),
        ),
    )(cat2d, int_feats, w1, b1, w2, b2, proj_w, proj_b, ln_g, ln_b)


def _packed_indices(int_feats):
    offsets = jnp.asarray(_COL_TABLE.astype(np.int64) * (4 * _QPS), dtype=jnp.int32)
    h = int_feats // _QPS
    p = int_feats - h * _QPS
    pk = offsets[None, :] + 4 * p + h
    return jnp.pad(pk, ((0, 0), (0, 128 - _NF)))


def kernel(int_feats, tables, w1, b1, w2, b2, proj_w, proj_b, ln_g, ln_b):
    t2 = tables.transpose(0, 2, 1)
    tab = _run_transpose(t2).reshape(26 * _QPS, 128)
    pk = _packed_indices(int_feats)
    cat_pk = _run_sc(pk, tab)
    return _run_tc(
        cat_pk.reshape(_B, _NSEG * _D), int_feats, w1,
        b1.reshape(1, 416), w2, b2.reshape(1, 1664),
        proj_w, proj_b.reshape(_NUM_TOKENS, 1, 256),
        ln_g.reshape(_NUM_TOKENS, 1, 256), ln_b.reshape(_NUM_TOKENS, 1, 256),
    )


# speedup vs baseline: 14.0947x; 1.0761x over previous
import functools

import jax
import jax.numpy as jnp
import numpy as np
from jax import lax
from jax.experimental import pallas as pl
from jax.experimental.pallas import tpu as pltpu
from jax.experimental.pallas import tpu_sc as plsc

_SPECS = [(100000, i, 1) for i in range(20)] + [(100000, 20 + 10 * j, 10) for j in range(6)]
_NUM_TOKENS = 8
_CHUNK = 208
_NF = 80
_NSEG = 26
_D = 64
_B = 4096
_NW = 32
_RPW = _B // _NW
_V = 100001
_QPS = 25088
_PCH = 12544

_COL_TABLE = np.zeros((_NF,), dtype=np.int32)
for _i, (_, _off, _ln) in enumerate(_SPECS):
    _COL_TABLE[_off:_off + _ln] = _i


def _tr_body(a_ref, b_ref, c_ref, d_ref, out_ref):
    f32 = jnp.float32
    r = lax.broadcasted_iota(jnp.int32, (256, 256), 0)
    c = lax.broadcasted_iota(jnp.int32, (256, 256), 1)
    cm = c % 128
    perm = ((r // _D == cm // 32)
            & (r % _D == cm % 32 + 32 * (c // 128))).astype(f32)
    s = jnp.concatenate([a_ref[0], b_ref[0], c_ref[0], d_ref[0]], axis=0)
    s = jnp.where(jnp.isfinite(s), s, 0.0)
    y = lax.dot_general(s, perm, (((0,), (0,)), ((), ())),
                        preferred_element_type=f32)
    blo = lax.bitcast_convert_type(y[:, :128], jnp.int32)
    bhi = lax.bitcast_convert_type(y[:, 128:], jnp.int32)
    rlo = blo + 0x7FFF + ((blo >> 16) & 1)
    rhi = bhi + 0x7FFF + ((bhi >> 16) & 1)
    out_ref[0] = ((rlo >> 16) & 0xFFFF) | (rhi & jnp.int32(-65536))


def _run_transpose(t2):
    grid_p = _QPS // _PCH
    return pl.pallas_call(
        _tr_body,
        grid=(26, grid_p),
        in_specs=[
            pl.BlockSpec((1, _D, _PCH), lambda i, c: (i, 0, c)),
            pl.BlockSpec((1, _D, _PCH), lambda i, c: (i, 0, c + grid_p)),
            pl.BlockSpec((1, _D, _PCH), lambda i, c: (i, 0, c + 2 * grid_p)),
            pl.BlockSpec((1, _D, _PCH), lambda i, c: (i, 0, c + 3 * grid_p)),
        ],
        out_specs=pl.BlockSpec((1, _PCH, 128), lambda i, c: (i, c, 0)),
        out_shape=jax.ShapeDtypeStruct((26, _QPS, 128), jnp.int32),
        compiler_params=pltpu.CompilerParams(
            dimension_semantics=("arbitrary", "arbitrary"),
        ),
    )(t2, t2, t2, t2)


def _sc_body(idx_hbm, tab_hbm, out_hbm, idx_v, pidx_v, win_v, obuf_v, sem, semo):
    wid = lax.axis_index("s") * 2 + lax.axis_index("c")
    b0 = wid * _RPW
    pltpu.sync_copy(idx_hbm.at[pl.ds(b0, _RPW)], idx_v)

    def issue(r, buf):
        for q in range(5):
            pidx_v[buf, pl.ds(16 * q, 16)] = idx_v[r, pl.ds(16 * q, 16)] >> 2
        pltpu.async_copy(tab_hbm.at[pidx_v.at[buf]], win_v.at[buf], sem)

    def wait_buf(buf):
        pltpu.make_async_copy(tab_hbm.at[pidx_v.at[buf]], win_v.at[buf], sem).wait()

    def halves(buf, c, h):
        base = h * 32
        lo0 = win_v[buf, c, pl.ds(base, 16)]
        lo1 = win_v[buf, c, pl.ds(base + 16, 16)]
        return [
            lax.bitcast_convert_type(lo0 << 16, jnp.float32),
            lax.bitcast_convert_type(lo1 << 16, jnp.float32),
            lax.bitcast_convert_type(lo0 & -65536, jnp.float32),
            lax.bitcast_convert_type(lo1 & -65536, jnp.float32),
        ]

    def process(r, rr, buf):
        hs = [idx_v[r, pl.ds(16 * q, 16)] & 3 for q in range(5)]
        for c in range(20):
            h = hs[c // 16][c % 16]
            vals = halves(buf, c, h)
            for m in range(4):
                obuf_v[(rr * 13) + c // 2, pl.ds((c % 2) * _D + 16 * m, 16)] = vals[m]
        for j in range(6):
            accs = [None] * 4
            for t in range(10):
                c = 20 + 10 * j + t
                h = hs[c // 16][c % 16]
                vals = halves(buf, c, h)
                for m in range(4):
                    accs[m] = vals[m] if accs[m] is None else accs[m] + vals[m]
            s = 20 + j
            for m in range(4):
                obuf_v[(rr * 13) + s // 2, pl.ds((s % 2) * _D + 16 * m, 16)] = accs[m]

    issue(0, 0)

    def blk_step(blk, _):
        r0 = blk * 8
        for gg in range(4):
            ra = r0 + 2 * gg
            rb = ra + 1
            issue(rb, 1)
            wait_buf(0)
            process(ra, 2 * gg, 0)
            nxt = jnp.minimum(ra + 2, _RPW - 1)
            issue(nxt, 0)
            wait_buf(1)
            process(rb, 2 * gg + 1, 1)
        pltpu.sync_copy(obuf_v, out_hbm.at[pl.ds((b0 + r0) * 13, 104)])
        return ()

    lax.fori_loop(0, _RPW // 8, blk_step, (), unroll=False)
    wait_buf(0)


def _run_sc(pk, tab):
    mesh = plsc.VectorSubcoreMesh(core_axis_name="c", subcore_axis_name="s")
    return pl.kernel(
        _sc_body,
        mesh=mesh,
        out_type=jax.ShapeDtypeStruct((_B * 13, 128), jnp.float32),
        scratch_types=[
            pltpu.VMEM((_RPW, 128), jnp.int32),
            pltpu.VMEM((2, _NF), jnp.int32),
            pltpu.VMEM((2, _NF, 128), jnp.int32),
            pltpu.VMEM((104, 128), jnp.float32),
            pltpu.SemaphoreType.DMA,
            pltpu.SemaphoreType.DMA,
        ],
    )(pk, tab)


_BT = 256


def _tc_body(cat_ref, intf_ref, w1_ref, b1_ref, w2_ref, b2_ref,
             pw_ref, pb_ref, lg_ref, lb_ref, out_ref):
    f32 = jnp.float32
    cat_sum = cat_ref[...]
    xi = intf_ref[...]
    nz = (xi != 0).astype(f32)
    c_iota = lax.broadcasted_iota(jnp.int32, (_NF, _NSEG), 0)
    s_iota = lax.broadcasted_iota(jnp.int32, (_NF, _NSEG), 1)
    H = ((s_iota >= 20) & (c_iota >= 10 * s_iota - 180)
         & (c_iota < 10 * s_iota - 170)).astype(f32)
    counts = jnp.dot(nz, H, preferred_element_type=f32)
    recip = 1.0 / jnp.maximum(counts, 1.0)
    seg_of = lax.broadcasted_iota(jnp.int32, (_NSEG, 1664), 1) // _D
    E = (seg_of == lax.broadcasted_iota(jnp.int32, (_NSEG, 1664), 0)).astype(f32)
    scale = jnp.dot(recip, E, preferred_element_type=f32)
    cat = cat_sum * scale

    h = jnp.dot(cat, w1_ref[...], preferred_element_type=f32) + b1_ref[...]
    h = h * jax.nn.sigmoid(h)
    gate = jax.nn.sigmoid(jnp.dot(h, w2_ref[...], preferred_element_type=f32) + b2_ref[...])
    cat = cat * gate * 2.0

    for t in range(_NUM_TOKENS):
        xt = cat[:, _CHUNK * t:_CHUNK * (t + 1)]
        y = jnp.dot(xt, pw_ref[t], preferred_element_type=f32) + pb_ref[t]
        mu = jnp.mean(y, axis=-1, keepdims=True)
        var = jnp.mean((y - mu) ** 2, axis=-1, keepdims=True)
        out_ref[:, t, :] = (y - mu) / jnp.sqrt(var + 1e-5) * lg_ref[t] + lb_ref[t]


def _run_tc(cat2d, int_feats, w1, b1, w2, b2, proj_w, proj_b, ln_g, ln_b):
    full = lambda shape: pl.BlockSpec(shape, lambda i: tuple(0 for _ in shape))
    return pl.pallas_call(
        _tc_body,
        grid=(_B // _BT,),
        in_specs=[
            pl.BlockSpec((_BT, _NSEG * _D), lambda i: (i, 0)),
            pl.BlockSpec((_BT, _NF), lambda i: (i, 0)),
            full((1664, 416)),
            full((1, 416)),
            full((416, 1664)),
            full((1, 1664)),
            full((_NUM_TOKENS, _CHUNK, 256)),
            full((_NUM_TOKENS, 1, 256)),
            full((_NUM_TOKENS, 1, 256)),
            full((_NUM_TOKENS, 1, 256)),
        ],
        out_specs=pl.BlockSpec((_BT, _NUM_TOKENS, 256), lambda i: (i, 0, 0)),
        out_shape=jax.ShapeDtypeStruct((_B, _NUM_TOKENS, 256), jnp.float32),
        compiler_params=pltpu.CompilerParams(
            dimension_semantics=("arbitrary",),
        ),
    )(cat2d, int_feats, w1, b1, w2, b2, proj_w, proj_b, ln_g, ln_b)


def _packed_indices(int_feats):
    offsets = jnp.asarray(_COL_TABLE.astype(np.int64) * (4 * _QPS), dtype=jnp.int32)
    h = int_feats // _QPS
    p = int_feats - h * _QPS
    pk = offsets[None, :] + 4 * p + h
    return jnp.pad(pk, ((0, 0), (0, 128 - _NF)))


def kernel(int_feats, tables, w1, b1, w2, b2, proj_w, proj_b, ln_g, ln_b):
    t2 = tables.transpose(0, 2, 1)
    tab = _run_transpose(t2).reshape(26 * _QPS, 128)
    pk = _packed_indices(int_feats)
    cat_pk = _run_sc(pk, tab)
    return _run_tc(
        cat_pk.reshape(_B, _NSEG * _D), int_feats, w1,
        b1.reshape(1, 416), w2, b2.reshape(1, 1664),
        proj_w, proj_b.reshape(_NUM_TOKENS, 1, 256),
        ln_g.reshape(_NUM_TOKENS, 1, 256), ln_b.reshape(_NUM_TOKENS, 1, 256),
    )
